# baseline (speedup 1.0000x reference)
import math

import jax
import jax.numpy as jnp
from jax.experimental import pallas as pl
from jax.experimental.pallas import tpu as pltpu

B = 1
S = 2048
D_MODEL = 768
H = 12
DH = D_MODEL // H
BS = 256
NB = S // BS
TOPK = 3
SCALE = 1.0 / math.sqrt(DH)
NEG = -1e30

_DN = (((1,), (1,)), ((), ()))


def _moba_kernel(xq_ref, xk_ref, xv_ref, wq_ref, bq_ref, wk_ref, bk_ref,
                 wv_ref, bv_ref, wo_ref, bo_ref, o_ref,
                 k_sc, v_sc, km_sc):
    m = pl.program_id(0)

    q = jax.lax.dot_general(xq_ref[:], wq_ref[:], _DN,
                            preferred_element_type=jnp.float32) + bq_ref[:]
    k = jax.lax.dot_general(xk_ref[:], wk_ref[:], _DN,
                            preferred_element_type=jnp.float32) + bk_ref[:]
    v = jax.lax.dot_general(xv_ref[:], wv_ref[:], _DN,
                            preferred_element_type=jnp.float32) + bv_ref[:]
    k_sc[pl.ds(m * BS, BS), :] = k.astype(jnp.bfloat16)
    v_sc[pl.ds(m * BS, BS), :] = v.astype(jnp.bfloat16)
    km_sc[pl.ds(m, 1), :] = jnp.mean(k, axis=0, keepdims=True)
    q_bf = q.astype(jnp.bfloat16)

    jidx = jax.lax.broadcasted_iota(jnp.int32, (BS, NB), 1)
    rows = jax.lax.broadcasted_iota(jnp.int32, (BS, BS), 0)
    cols = jax.lax.broadcasted_iota(jnp.int32, (BS, BS), 1)

    outs = []
    for h in range(H):
        lo = h * DH
        qh = q[:, lo:lo + DH]
        qh_bf = q_bf[:, lo:lo + DH]
        km = km_sc[:, lo:lo + DH]

        gate = jax.lax.dot_general(qh, km, _DN,
                                   preferred_element_type=jnp.float32)
        gate = jnp.where(jidx < m, gate, NEG)
        rank = jnp.zeros((BS, NB), jnp.int32)
        for jp in range(NB):
            gp = gate[:, jp:jp + 1]
            ahead = (gp > gate) | ((gp == gate) & (jp < jidx))
            rank = rank + ahead.astype(jnp.int32)
        sel = ((rank < TOPK) & (jidx < m)).astype(jnp.float32)

        k_i = k_sc[pl.ds(m * BS, BS), lo:lo + DH]
        v_i = v_sc[pl.ds(m * BS, BS), lo:lo + DH]
        s = jax.lax.dot_general(qh_bf, k_i, _DN,
                                preferred_element_type=jnp.float32) * SCALE
        s = jnp.where(cols <= rows, s, NEG)
        m_self = jnp.max(s, axis=1, keepdims=True)
        p = jnp.exp(s - m_self)
        l_self = jnp.sum(p, axis=1, keepdims=True)
        o_self = jax.lax.dot_general(
            p.astype(jnp.bfloat16), v_i, (((1,), (0,)), ((), ())),
            preferred_element_type=jnp.float32) / l_self

        def body(j, carry, _lo=lo, _sel=sel, _qh=qh_bf):
            mx, l, acc = carry
            k_j = k_sc[pl.ds(j * BS, BS), _lo:_lo + DH]
            v_j = v_sc[pl.ds(j * BS, BS), _lo:_lo + DH]
            sj = jax.lax.dot_general(_qh, k_j, _DN,
                                     preferred_element_type=jnp.float32) * SCALE
            sel_j = jnp.sum(jnp.where(jidx == j, _sel, 0.0), axis=1,
                            keepdims=True)
            sj = jnp.where(sel_j > 0.0, sj, NEG)
            m_new = jnp.maximum(mx, jnp.max(sj, axis=1, keepdims=True))
            alpha = jnp.exp(mx - m_new)
            pj = jnp.exp(sj - m_new) * sel_j
            l = l * alpha + jnp.sum(pj, axis=1, keepdims=True)
            acc = acc * alpha + jax.lax.dot_general(
                pj.astype(jnp.bfloat16), v_j, (((1,), (0,)), ((), ())),
                preferred_element_type=jnp.float32)
            return m_new, l, acc

        m0 = jnp.full((BS, 1), NEG, jnp.float32)
        l0 = jnp.zeros((BS, 1), jnp.float32)
        a0 = jnp.zeros((BS, DH), jnp.float32)
        _, l, acc = jax.lax.fori_loop(0, m, body, (m0, l0, a0))
        o_moba = jnp.where(l > 0.0, acc / jnp.maximum(l, 1e-30), 0.0)

        outs.append(o_self + o_moba)

    combined = jnp.concatenate(outs, axis=1)
    o_ref[:] = jax.lax.dot_general(
        combined.astype(jnp.bfloat16), wo_ref[:], _DN,
        preferred_element_type=jnp.float32) + bo_ref[:]


def kernel(query, key, value, Wq, bq, Wk, bk, Wv, bv, Wo, bo):
    xq = query.reshape(S, D_MODEL)
    xk = key.reshape(S, D_MODEL)
    xv = value.reshape(S, D_MODEL)

    row_spec = pl.BlockSpec((BS, D_MODEL), lambda mm: (mm, 0))
    w_spec = pl.BlockSpec((D_MODEL, D_MODEL), lambda mm: (0, 0))
    b_spec = pl.BlockSpec((1, D_MODEL), lambda mm: (0, 0))

    out = pl.pallas_call(
        _moba_kernel,
        grid=(NB,),
        in_specs=[row_spec, row_spec, row_spec,
                  w_spec, b_spec, w_spec, b_spec, w_spec, b_spec,
                  w_spec, b_spec],
        out_specs=row_spec,
        out_shape=jax.ShapeDtypeStruct((S, D_MODEL), jnp.float32),
        scratch_shapes=[
            pltpu.VMEM((S, D_MODEL), jnp.bfloat16),
            pltpu.VMEM((S, D_MODEL), jnp.bfloat16),
            pltpu.VMEM((NB, D_MODEL), jnp.float32),
        ],
    )(xq, xk, xv.astype(jnp.bfloat16), Wq, bq.reshape(1, -1),
      Wk, bk.reshape(1, -1), Wv.astype(jnp.bfloat16), bv.reshape(1, -1),
      Wo.astype(jnp.bfloat16), bo.reshape(1, -1))

    return out.reshape(B, S, D_MODEL)


# speedup vs baseline: 1.4319x; 1.4319x over previous
import math

import jax
import jax.numpy as jnp
from jax.experimental import pallas as pl
from jax.experimental.pallas import tpu as pltpu

B = 1
S = 2048
D_MODEL = 768
H = 12
DH = D_MODEL // H
BS = 256
NB = S // BS
TOPK = 3
SCALE = 1.0 / math.sqrt(DH)
NEG = -1e30

_DN = (((1,), (1,)), ((), ()))
_DN00 = (((0,), (0,)), ((), ()))
_DN01 = (((0,), (1,)), ((), ()))


def _moba_kernel(xq_ref, xk_ref, xv_ref, wq_ref, bq_ref, wk_ref, bk_ref,
                 wv_ref, bv_ref, wo_ref, bo_ref, o_ref,
                 k_sc, v_sc, km_sc):
    m = pl.program_id(0)

    q = jax.lax.dot_general(xq_ref[:], wq_ref[:], _DN,
                            preferred_element_type=jnp.float32) + bq_ref[:]
    k = jax.lax.dot_general(xk_ref[:], wk_ref[:], _DN,
                            preferred_element_type=jnp.float32) + bk_ref[:]
    v = jax.lax.dot_general(xv_ref[:], wv_ref[:], _DN,
                            preferred_element_type=jnp.float32) + bv_ref[:]
    k_sc[pl.ds(m * BS, BS), :] = k
    v_sc[pl.ds(m * BS, BS), :] = v
    km_sc[pl.ds(m, 1), :] = jnp.mean(k, axis=0, keepdims=True)

    jidx = jax.lax.broadcasted_iota(jnp.int32, (NB, BS), 0)
    rows = jax.lax.broadcasted_iota(jnp.int32, (BS, BS), 0)
    cols = jax.lax.broadcasted_iota(jnp.int32, (BS, BS), 1)

    outs = []
    for h in range(H):
        lo = h * DH
        qh = q[:, lo:lo + DH]
        km = km_sc[:, lo:lo + DH]

        gate = jax.lax.dot_general(km, qh, _DN,
                                   preferred_element_type=jnp.float32)
        gate = jnp.where(jidx < m, gate, NEG)
        rank = jnp.zeros((NB, BS), jnp.int32)
        for jp in range(NB):
            gp = gate[jp:jp + 1, :]
            ahead = (gp > gate) | ((gp == gate) & (jp < jidx))
            rank = rank + ahead.astype(jnp.int32)
        sel = ((rank < TOPK) & (jidx < m)).astype(jnp.float32)

        k_i = k_sc[pl.ds(m * BS, BS), lo:lo + DH]
        v_i = v_sc[pl.ds(m * BS, BS), lo:lo + DH]
        s = jax.lax.dot_general(k_i, qh, _DN,
                                preferred_element_type=jnp.float32) * SCALE
        s = jnp.where(rows <= cols, s, NEG)
        m_self = jnp.max(s, axis=0, keepdims=True)
        p = jnp.exp(s - m_self)
        l_self = jnp.sum(p, axis=0, keepdims=True)
        o_self = jax.lax.dot_general(
            v_i, p, _DN00, preferred_element_type=jnp.float32) / l_self

        def body(j, carry, _lo=lo, _sel=sel, _qh=qh):
            mx, l, acc = carry
            k_j = k_sc[pl.ds(j * BS, BS), _lo:_lo + DH]
            v_j = v_sc[pl.ds(j * BS, BS), _lo:_lo + DH]
            sj = jax.lax.dot_general(k_j, _qh, _DN,
                                     preferred_element_type=jnp.float32) * SCALE
            sel_j = jnp.sum(jnp.where(jidx == j, _sel, 0.0), axis=0,
                            keepdims=True)
            sj = jnp.where(sel_j > 0.0, sj, NEG)
            m_new = jnp.maximum(mx, jnp.max(sj, axis=0, keepdims=True))
            alpha = jnp.exp(mx - m_new)
            pj = jnp.exp(sj - m_new) * sel_j
            l = l * alpha + jnp.sum(pj, axis=0, keepdims=True)
            acc = acc * alpha + jax.lax.dot_general(
                v_j, pj, _DN00, preferred_element_type=jnp.float32)
            return m_new, l, acc

        m0 = jnp.full((1, BS), NEG, jnp.float32)
        l0 = jnp.zeros((1, BS), jnp.float32)
        a0 = jnp.zeros((DH, BS), jnp.float32)
        _, l, acc = jax.lax.fori_loop(0, m, body, (m0, l0, a0))
        o_moba = jnp.where(l > 0.0, acc / jnp.maximum(l, 1e-30), 0.0)

        outs.append(o_self + o_moba)

    combined = jnp.concatenate(outs, axis=0)
    o_ref[:] = jax.lax.dot_general(
        combined, wo_ref[:], _DN01,
        preferred_element_type=jnp.float32) + bo_ref[:]


def kernel(query, key, value, Wq, bq, Wk, bk, Wv, bv, Wo, bo):
    xq = query.reshape(S, D_MODEL)
    xk = key.reshape(S, D_MODEL)
    xv = value.reshape(S, D_MODEL)

    row_spec = pl.BlockSpec((BS, D_MODEL), lambda mm: (mm, 0))
    w_spec = pl.BlockSpec((D_MODEL, D_MODEL), lambda mm: (0, 0))
    b_spec = pl.BlockSpec((1, D_MODEL), lambda mm: (0, 0))

    out = pl.pallas_call(
        _moba_kernel,
        grid=(NB,),
        in_specs=[row_spec, row_spec, row_spec,
                  w_spec, b_spec, w_spec, b_spec, w_spec, b_spec,
                  w_spec, b_spec],
        out_specs=row_spec,
        out_shape=jax.ShapeDtypeStruct((S, D_MODEL), jnp.float32),
        scratch_shapes=[
            pltpu.VMEM((S, D_MODEL), jnp.float32),
            pltpu.VMEM((S, D_MODEL), jnp.float32),
            pltpu.VMEM((NB, D_MODEL), jnp.float32),
        ],
    )(xq, xk, xv, Wq, bq.reshape(1, -1), Wk, bk.reshape(1, -1),
      Wv, bv.reshape(1, -1), Wo, bo.reshape(1, -1))

    return out.reshape(B, S, D_MODEL)


# speedup vs baseline: 1.9017x; 1.3281x over previous
import math

import jax
import jax.numpy as jnp
from jax.experimental import pallas as pl
from jax.experimental.pallas import tpu as pltpu

B = 1
S = 2048
D_MODEL = 768
H = 12
DH = D_MODEL // H
BS = 256
NB = S // BS
TOPK = 3
SCALE = 1.0 / math.sqrt(DH)
NEG = -1e30

_DN = (((1,), (1,)), ((), ()))
_DN00 = (((0,), (0,)), ((), ()))
_DN01 = (((0,), (1,)), ((), ()))


def _moba_kernel(xq_ref, xk_ref, xv_ref, wq_ref, bq_ref, wk_ref, bk_ref,
                 wv_ref, bv_ref, wo_ref, bo_ref, o_ref,
                 k_sc, v_sc, km_sc):
    m = pl.program_id(0)

    @pl.when(m == 0)
    def _init():
        v_sc[:] = jnp.zeros((S, D_MODEL), jnp.float32)

    q = jax.lax.dot_general(xq_ref[:], wq_ref[:], _DN,
                            preferred_element_type=jnp.float32) + bq_ref[:]
    k = jax.lax.dot_general(xk_ref[:], wk_ref[:], _DN,
                            preferred_element_type=jnp.float32) + bk_ref[:]
    v = jax.lax.dot_general(xv_ref[:], wv_ref[:], _DN,
                            preferred_element_type=jnp.float32) + bv_ref[:]
    k_sc[pl.ds(m * BS, BS), :] = k
    v_sc[pl.ds(m * BS, BS), :] = v
    km_sc[pl.ds(m, 1), :] = jnp.mean(k, axis=0, keepdims=True)

    jidx = jax.lax.broadcasted_iota(jnp.int32, (NB, BS), 0)
    rows = jax.lax.broadcasted_iota(jnp.int32, (BS, BS), 0)
    cols = jax.lax.broadcasted_iota(jnp.int32, (BS, BS), 1)

    outs = []
    for h in range(H):
        lo = h * DH
        qh = q[:, lo:lo + DH]
        km = km_sc[:, lo:lo + DH]

        gate = jax.lax.dot_general(km, qh, _DN,
                                   preferred_element_type=jnp.float32)
        gate = jnp.where(jidx < m, gate, NEG)
        rank = jnp.zeros((NB, BS), jnp.int32)
        for jp in range(NB):
            gp = gate[jp:jp + 1, :]
            ahead = (gp > gate) | ((gp == gate) & (jp < jidx))
            rank = rank + ahead.astype(jnp.int32)
        sel = ((rank < TOPK) & (jidx < m)).astype(jnp.float32)

        k_i = k_sc[pl.ds(m * BS, BS), lo:lo + DH]
        v_i = v_sc[pl.ds(m * BS, BS), lo:lo + DH]
        s = jax.lax.dot_general(k_i, qh, _DN,
                                preferred_element_type=jnp.float32) * SCALE
        s = jnp.where(rows <= cols, s, NEG)
        m_self = jnp.max(s, axis=0, keepdims=True)
        p = jnp.exp(s - m_self)
        l_self = jnp.sum(p, axis=0, keepdims=True)
        o_self = jax.lax.dot_general(
            v_i, p, _DN00, preferred_element_type=jnp.float32) / l_self

        k_pri = k_sc[0:(NB - 1) * BS, lo:lo + DH]
        v_pri = v_sc[0:(NB - 1) * BS, lo:lo + DH]
        s_all = jax.lax.dot_general(k_pri, qh, _DN,
                                    preferred_element_type=jnp.float32) * SCALE
        subs = [jnp.where(sel[j:j + 1, :] > 0.0,
                          s_all[j * BS:(j + 1) * BS, :], NEG)
                for j in range(NB - 1)]
        m_moba = jnp.full((1, BS), NEG, jnp.float32)
        for sub in subs:
            m_moba = jnp.maximum(m_moba, jnp.max(sub, axis=0, keepdims=True))
        p_subs = [jnp.exp(sub - m_moba) * sel[j:j + 1, :]
                  for j, sub in enumerate(subs)]
        p_all = jnp.concatenate(p_subs, axis=0)
        l = jnp.sum(p_all, axis=0, keepdims=True)
        acc = jax.lax.dot_general(v_pri, p_all, _DN00,
                                  preferred_element_type=jnp.float32)
        o_moba = jnp.where(l > 0.0, acc / jnp.maximum(l, 1e-30), 0.0)

        outs.append(o_self + o_moba)

    combined = jnp.concatenate(outs, axis=0)
    o_ref[:] = jax.lax.dot_general(
        combined, wo_ref[:], _DN01,
        preferred_element_type=jnp.float32) + bo_ref[:]


def kernel(query, key, value, Wq, bq, Wk, bk, Wv, bv, Wo, bo):
    xq = query.reshape(S, D_MODEL)
    xk = key.reshape(S, D_MODEL)
    xv = value.reshape(S, D_MODEL)

    row_spec = pl.BlockSpec((BS, D_MODEL), lambda mm: (mm, 0))
    w_spec = pl.BlockSpec((D_MODEL, D_MODEL), lambda mm: (0, 0))
    b_spec = pl.BlockSpec((1, D_MODEL), lambda mm: (0, 0))

    out = pl.pallas_call(
        _moba_kernel,
        grid=(NB,),
        in_specs=[row_spec, row_spec, row_spec,
                  w_spec, b_spec, w_spec, b_spec, w_spec, b_spec,
                  w_spec, b_spec],
        out_specs=row_spec,
        out_shape=jax.ShapeDtypeStruct((S, D_MODEL), jnp.float32),
        scratch_shapes=[
            pltpu.VMEM((S, D_MODEL), jnp.float32),
            pltpu.VMEM((S, D_MODEL), jnp.float32),
            pltpu.VMEM((NB, D_MODEL), jnp.float32),
        ],
    )(xq, xk, xv, Wq, bq.reshape(1, -1), Wk, bk.reshape(1, -1),
      Wv, bv.reshape(1, -1), Wo, bo.reshape(1, -1))

    return out.reshape(B, S, D_MODEL)
